# baseline (speedup 1.0000x reference)
import jax
import jax.numpy as jnp
from jax.experimental import pallas as pl
from jax.experimental.pallas import tpu as pltpu

NUM_SEGMENTS = 1024
BLK = 10000
CH = 1000
NCH = BLK // CH
W_WIN = 24

_PREC = jax.lax.Precision.DEFAULT


def _fused_kernel(firsts_ref, lasts_ref,
                  x_ref, idx_ref,
                  w1_ref, b1_ref, w2_ref, b2_ref,
                  wr1_ref, br1_ref, wr2_ref, br2_ref,
                  out_ref, acc_ref):
    g = pl.program_id(0)
    nblk = pl.num_programs(0)

    @pl.when(g == 0)
    def _init():
        acc_ref[...] = jnp.zeros_like(acc_ref)

    xb = x_ref[...]
    h = jnp.maximum(jnp.dot(xb, w1_ref[...], precision=_PREC) + b1_ref[...], 0.0)
    xp = jnp.dot(h, w2_ref[...], precision=_PREC) + b2_ref[...]
    xpb = xp.astype(jnp.bfloat16)

    idxm = idx_ref[0]
    iota = jax.lax.broadcasted_iota(jnp.int32, (W_WIN, CH), 0)

    for c in range(NCH):
        idc = idxm[c:c + 1, :]
        xpc = xpb[c * CH:(c + 1) * CH, :]
        last = lasts_ref[g, c]

        def _body(k8, idc=idc, xpc=xpc):
            base = k8 * 8
            rel = idc - base
            oh_t = (rel == iota).astype(jnp.bfloat16)
            contrib = jnp.dot(oh_t, xpc, preferred_element_type=jnp.float32)
            acc_ref[pl.ds(base, W_WIN), :] += contrib
            return k8 + W_WIN // 8

        def _cond(k8, last=last):
            return k8 * 8 <= last

        jax.lax.while_loop(_cond, _body, firsts_ref[g, c] // 8)

    @pl.when(g == nblk - 1)
    def _rho():
        xs = acc_ref[:NUM_SEGMENTS, :]
        h2 = jnp.maximum(jnp.dot(xs, wr1_ref[...], precision=_PREC) + br1_ref[...], 0.0)
        out_ref[...] = jnp.dot(h2, wr2_ref[...], precision=_PREC) + br2_ref[...]


def kernel(x, batch_idx, W_phi1, b_phi1, W_phi2, b_phi2, W_rho1, b_rho1, W_rho2, b_rho2):
    n, d_in = x.shape
    d_out = W_rho2.shape[1]
    assert n % BLK == 0
    nblk = n // BLK

    idx = batch_idx.astype(jnp.int32)
    idx3 = idx.reshape(nblk, NCH, CH)
    firsts = idx[::CH].reshape(nblk, NCH)
    lasts = idx[CH - 1::CH].reshape(nblk, NCH)

    b1 = b_phi1.reshape(1, -1)
    b2 = b_phi2.reshape(1, -1)
    br1 = b_rho1.reshape(1, -1)
    br2 = b_rho2.reshape(1, -1)

    const = lambda *_: (0, 0)
    grid_spec = pltpu.PrefetchScalarGridSpec(
        num_scalar_prefetch=2,
        grid=(nblk,),
        in_specs=[
            pl.BlockSpec((BLK, d_in), lambda g, f, l: (g, 0)),
            pl.BlockSpec((1, NCH, CH), lambda g, f, l: (g, 0, 0)),
            pl.BlockSpec(W_phi1.shape, const),
            pl.BlockSpec(b1.shape, const),
            pl.BlockSpec(W_phi2.shape, const),
            pl.BlockSpec(b2.shape, const),
            pl.BlockSpec(W_rho1.shape, const),
            pl.BlockSpec(br1.shape, const),
            pl.BlockSpec(W_rho2.shape, const),
            pl.BlockSpec(br2.shape, const),
        ],
        out_specs=pl.BlockSpec((NUM_SEGMENTS, d_out), const),
        scratch_shapes=[pltpu.VMEM((NUM_SEGMENTS + W_WIN, d_in), jnp.float32)],
    )

    return pl.pallas_call(
        _fused_kernel,
        grid_spec=grid_spec,
        out_shape=jax.ShapeDtypeStruct((NUM_SEGMENTS, d_out), jnp.float32),
        compiler_params=pltpu.CompilerParams(
            dimension_semantics=("arbitrary",),
        ),
    )(firsts, lasts, x, idx3, W_phi1, b1, W_phi2, b2, W_rho1, br1, W_rho2, br2)


# speedup vs baseline: 1.2623x; 1.2623x over previous
import jax
import jax.numpy as jnp
from jax.experimental import pallas as pl
from jax.experimental.pallas import tpu as pltpu

NUM_SEGMENTS = 1024
BLK = 10000
W_WIN = 128

_PREC = jax.lax.Precision.DEFAULT


def _fused_kernel(firsts_ref, lasts_ref,
                  x_ref, idx_ref,
                  w1_ref, b1_ref, w2_ref, b2_ref,
                  wr1_ref, br1_ref, wr2_ref, br2_ref,
                  out_ref, acc_ref):
    g = pl.program_id(0)
    nblk = pl.num_programs(0)

    @pl.when(g == 0)
    def _init():
        acc_ref[...] = jnp.zeros_like(acc_ref)

    xb = x_ref[...]
    h = jnp.maximum(jnp.dot(xb, w1_ref[...], precision=_PREC) + b1_ref[...], 0.0)
    xp = jnp.dot(h, w2_ref[...], precision=_PREC) + b2_ref[...]
    xpb = xp.astype(jnp.bfloat16)

    idxv = idx_ref[0]
    last = lasts_ref[g]
    iota = jax.lax.broadcasted_iota(jnp.int32, (W_WIN, BLK), 0)

    def _cond(k8):
        return k8 * 8 <= last

    def _body(k8):
        base = k8 * 8
        rel = idxv - base
        oh_t = (rel == iota).astype(jnp.bfloat16)
        contrib = jnp.dot(oh_t, xpb, preferred_element_type=jnp.float32)
        acc_ref[pl.ds(base, W_WIN), :] += contrib
        return k8 + W_WIN // 8

    jax.lax.while_loop(_cond, _body, firsts_ref[g] // 8)

    @pl.when(g == nblk - 1)
    def _rho():
        xs = acc_ref[:NUM_SEGMENTS, :]
        h2 = jnp.maximum(jnp.dot(xs, wr1_ref[...], precision=_PREC) + br1_ref[...], 0.0)
        out_ref[...] = jnp.dot(h2, wr2_ref[...], precision=_PREC) + br2_ref[...]


def kernel(x, batch_idx, W_phi1, b_phi1, W_phi2, b_phi2, W_rho1, b_rho1, W_rho2, b_rho2):
    n, d_in = x.shape
    d_out = W_rho2.shape[1]
    assert n % BLK == 0
    nblk = n // BLK

    idx = batch_idx.astype(jnp.int32)
    idx3 = idx.reshape(nblk, 1, BLK)
    firsts = idx[::BLK]
    lasts = idx[BLK - 1::BLK]

    b1 = b_phi1.reshape(1, -1)
    b2 = b_phi2.reshape(1, -1)
    br1 = b_rho1.reshape(1, -1)
    br2 = b_rho2.reshape(1, -1)

    const = lambda *_: (0, 0)
    grid_spec = pltpu.PrefetchScalarGridSpec(
        num_scalar_prefetch=2,
        grid=(nblk,),
        in_specs=[
            pl.BlockSpec((BLK, d_in), lambda g, f, l: (g, 0)),
            pl.BlockSpec((1, 1, BLK), lambda g, f, l: (g, 0, 0)),
            pl.BlockSpec(W_phi1.shape, const),
            pl.BlockSpec(b1.shape, const),
            pl.BlockSpec(W_phi2.shape, const),
            pl.BlockSpec(b2.shape, const),
            pl.BlockSpec(W_rho1.shape, const),
            pl.BlockSpec(br1.shape, const),
            pl.BlockSpec(W_rho2.shape, const),
            pl.BlockSpec(br2.shape, const),
        ],
        out_specs=pl.BlockSpec((NUM_SEGMENTS, d_out), const),
        scratch_shapes=[pltpu.VMEM((NUM_SEGMENTS + W_WIN, d_in), jnp.float32)],
    )

    return pl.pallas_call(
        _fused_kernel,
        grid_spec=grid_spec,
        out_shape=jax.ShapeDtypeStruct((NUM_SEGMENTS, d_out), jnp.float32),
        compiler_params=pltpu.CompilerParams(
            dimension_semantics=("arbitrary",),
        ),
    )(firsts, lasts, x, idx3, W_phi1, b1, W_phi2, b2, W_rho1, br1, W_rho2, br2)


# speedup vs baseline: 1.2827x; 1.0162x over previous
import jax
import jax.numpy as jnp
from jax.experimental import pallas as pl
from jax.experimental.pallas import tpu as pltpu

NUM_SEGMENTS = 1024
BLK = 10000
W_WIN = 128

_PREC = jax.lax.Precision.DEFAULT


def _fused_kernel(firsts_ref, lasts_ref,
                  x_ref, idx_ref,
                  w1_ref, b1_ref, w2_ref, b2_ref,
                  wr1_ref, br1_ref, wr2_ref, br2_ref,
                  out_ref, acc_ref):
    g = pl.program_id(0)
    nblk = pl.num_programs(0)

    @pl.when(g == 0)
    def _init():
        acc_ref[...] = jnp.zeros_like(acc_ref)

    idxv = idx_ref[0]
    last = lasts_ref[g]
    iota = jax.lax.broadcasted_iota(jnp.int32, (W_WIN, BLK), 0)

    k80 = firsts_ref[g] // 8
    base0 = k80 * 8
    oh0 = ((idxv - base0) == iota).astype(jnp.bfloat16)

    xb = x_ref[...]
    h = jnp.maximum(jnp.dot(xb, w1_ref[...], precision=_PREC) + b1_ref[...], 0.0)
    xp = jnp.dot(h, w2_ref[...], precision=_PREC) + b2_ref[...]
    xpb = xp.astype(jnp.bfloat16)

    contrib0 = jnp.dot(oh0, xpb, preferred_element_type=jnp.float32)
    acc_ref[pl.ds(base0, W_WIN), :] += contrib0

    def _cond(k8):
        return k8 * 8 <= last

    def _body(k8):
        base = k8 * 8
        rel = idxv - base
        oh_t = (rel == iota).astype(jnp.bfloat16)
        contrib = jnp.dot(oh_t, xpb, preferred_element_type=jnp.float32)
        acc_ref[pl.ds(base, W_WIN), :] += contrib
        return k8 + W_WIN // 8

    jax.lax.while_loop(_cond, _body, k80 + W_WIN // 8)

    @pl.when(g == nblk - 1)
    def _rho():
        xs = acc_ref[:NUM_SEGMENTS, :]
        h2 = jnp.maximum(jnp.dot(xs, wr1_ref[...], precision=_PREC) + br1_ref[...], 0.0)
        out_ref[...] = jnp.dot(h2, wr2_ref[...], precision=_PREC) + br2_ref[...]


def kernel(x, batch_idx, W_phi1, b_phi1, W_phi2, b_phi2, W_rho1, b_rho1, W_rho2, b_rho2):
    n, d_in = x.shape
    d_out = W_rho2.shape[1]
    assert n % BLK == 0
    nblk = n // BLK

    idx = batch_idx.astype(jnp.int32)
    idx3 = idx.reshape(nblk, 1, BLK)
    firsts = idx[::BLK]
    lasts = idx[BLK - 1::BLK]

    b1 = b_phi1.reshape(1, -1)
    b2 = b_phi2.reshape(1, -1)
    br1 = b_rho1.reshape(1, -1)
    br2 = b_rho2.reshape(1, -1)

    const = lambda *_: (0, 0)
    grid_spec = pltpu.PrefetchScalarGridSpec(
        num_scalar_prefetch=2,
        grid=(nblk,),
        in_specs=[
            pl.BlockSpec((BLK, d_in), lambda g, f, l: (g, 0)),
            pl.BlockSpec((1, 1, BLK), lambda g, f, l: (g, 0, 0)),
            pl.BlockSpec(W_phi1.shape, const),
            pl.BlockSpec(b1.shape, const),
            pl.BlockSpec(W_phi2.shape, const),
            pl.BlockSpec(b2.shape, const),
            pl.BlockSpec(W_rho1.shape, const),
            pl.BlockSpec(br1.shape, const),
            pl.BlockSpec(W_rho2.shape, const),
            pl.BlockSpec(br2.shape, const),
        ],
        out_specs=pl.BlockSpec((NUM_SEGMENTS, d_out), const),
        scratch_shapes=[pltpu.VMEM((NUM_SEGMENTS + W_WIN, d_in), jnp.float32)],
    )

    return pl.pallas_call(
        _fused_kernel,
        grid_spec=grid_spec,
        out_shape=jax.ShapeDtypeStruct((NUM_SEGMENTS, d_out), jnp.float32),
        compiler_params=pltpu.CompilerParams(
            dimension_semantics=("arbitrary",),
        ),
    )(firsts, lasts, x, idx3, W_phi1, b1, W_phi2, b2, W_rho1, br1, W_rho2, br2)
